# Initial kernel scaffold; baseline (speedup 1.0000x reference)
import jax
import jax.numpy as jnp
from jax.experimental import pallas as pl


def kernel(position_idx, table):
    raise NotImplementedError("write your pallas kernel here")


# speedup vs baseline: 6.6393x; 6.6393x over previous
import jax
import jax.numpy as jnp
from jax.experimental import pallas as pl
from jax.experimental.pallas import tpu as pltpu
from jax.experimental.pallas import tpu_sc as plsc

_WINDOW = 128


def kernel(position_idx, table):
    batch, seq = position_idx.shape
    n = batch * seq
    dim = table.shape[1]
    idx = position_idx.reshape(1, n)

    mesh = plsc.VectorSubcoreMesh(core_axis_name="core",
                                  subcore_axis_name="subcore")

    @jax.jit
    def run(table_arr, idx_arr):
        @pl.kernel(out_type=jax.ShapeDtypeStruct((n, dim), table_arr.dtype),
                   mesh=mesh,
                   compiler_params=pltpu.CompilerParams(
                       use_tc_tiling_on_sc=False))
        def gather_kernel(table_hbm, idx_hbm, out_hbm):
            def body(i_vmem, o_vmem):
                pltpu.sync_copy(table_hbm.at[i_vmem.at[0]], o_vmem)

            pltpu.emit_pipeline(
                body,
                grid=(n // _WINDOW,),
                in_specs=[pl.BlockSpec((1, _WINDOW),
                                       index_map=lambda i: (0, i))],
                out_specs=[pl.BlockSpec((_WINDOW, dim),
                                        index_map=lambda i: (i, 0))],
                core_axis_name=("core", "subcore"),
                dimension_semantics=(pltpu.PARALLEL,),
            )(idx_hbm, out_hbm)

        return gather_kernel(table_arr, idx_arr)

    return run(table, idx).reshape(batch, seq, dim)
